# baseline (speedup 1.0000x reference)
import functools

import jax
import jax.numpy as jnp
from jax import lax
from jax.experimental import pallas as pl
from jax.experimental.pallas import tpu as pltpu

BATCH = 4096
CORPUS = 100000
DIM = 128
K = 100

QT = 256
CT = 2048
CPAD = 100352
NBLK = CPAD // 128


def _matmul_body(q_ref, c_ref, s_ref, m_ref):
    j = pl.program_id(1)
    q = q_ref[...]
    c = c_ref[...]
    s = lax.dot_general(q, c, (((1,), (1,)), ((), ())),
                        preferred_element_type=jnp.float32)
    col0 = j * CT
    cols = col0 + lax.broadcasted_iota(jnp.int32, (QT, CT), 1)
    s = jnp.where(cols < CORPUS, s, -jnp.inf)
    s_ref[...] = s
    parts = [jnp.max(s[:, t * 128:(t + 1) * 128], axis=1, keepdims=True)
             for t in range(CT // 128)]
    m_ref[...] = jnp.concatenate(parts, axis=1)[None]


def _scores_and_blockmax(q, corpus_padded):
    grid = (BATCH // QT, CPAD // CT)
    return pl.pallas_call(
        _matmul_body,
        grid=grid,
        in_specs=[
            pl.BlockSpec((QT, DIM), lambda i, j: (i, 0)),
            pl.BlockSpec((CT, DIM), lambda i, j: (j, 0)),
        ],
        out_specs=[
            pl.BlockSpec((QT, CT), lambda i, j: (i, j)),
            pl.BlockSpec((1, QT, CT // 128), lambda i, j: (j, i, 0)),
        ],
        out_shape=[
            jax.ShapeDtypeStruct((BATCH, CPAD), jnp.float32),
            jax.ShapeDtypeStruct((CPAD // CT, BATCH, CT // 128), jnp.float32),
        ],
    )(q, corpus_padded)


def kernel(query_embedding, corpus, num_items):
    corpus_padded = jnp.pad(corpus, ((0, CPAD - CORPUS), (0, 0)))
    scores, _m1 = _scores_and_blockmax(query_embedding, corpus_padded)
    scores = scores[:, :CORPUS]
    mips_scores, indices = jax.lax.top_k(scores, K)
    indices = indices + (num_items - K)
    embeddings = corpus[indices]
    return (indices, mips_scores, embeddings)


# speedup vs baseline: 15.3572x; 15.3572x over previous
import functools

import jax
import jax.numpy as jnp
from jax import lax
from jax.experimental import pallas as pl
from jax.experimental.pallas import tpu as pltpu
from jax.experimental.pallas import tpu_sc as plsc

BATCH = 4096
CORPUS = 100000
DIM = 128
K = 100

QT = 256
CT = 2048
CPAD = 100352
NBLK = CPAD // 128

NW = 32
ROWS_PER_W = BATCH // NW

NCAND = 512
QT_D = 128

NEG_INF = float("-inf")


def _matmul_body(q_ref, c_ref, s_ref, m_ref):
    j = pl.program_id(1)
    q = q_ref[...]
    c = c_ref[...]
    s = lax.dot_general(q, c, (((1,), (1,)), ((), ())),
                        preferred_element_type=jnp.float32)
    cols = j * CT + lax.broadcasted_iota(jnp.int32, (QT, CT), 1)
    s = jnp.where(cols < CORPUS, s, NEG_INF)
    s_ref[...] = s
    parts = [jnp.max(s[:, t * 128:(t + 1) * 128], axis=1, keepdims=True)
             for t in range(CT // 128)]
    m_ref[...] = jnp.concatenate(parts, axis=1)[None]


def _scores_and_blockmax(q, corpus_padded):
    grid = (BATCH // QT, CPAD // CT)
    return pl.pallas_call(
        _matmul_body,
        grid=grid,
        in_specs=[
            pl.BlockSpec((QT, DIM), lambda i, j: (i, 0)),
            pl.BlockSpec((CT, DIM), lambda i, j: (j, 0)),
        ],
        out_specs=[
            pl.BlockSpec((QT, CT), lambda i, j: (i, j)),
            pl.BlockSpec((1, QT, CT // 128), lambda i, j: (j, i, 0)),
        ],
        out_shape=[
            jax.ShapeDtypeStruct((BATCH, CPAD), jnp.float32),
            jax.ShapeDtypeStruct((CPAD // CT, BATCH, CT // 128), jnp.float32),
        ],
    )(q, corpus_padded)


def _thr_body(m1_ref, thr_ref):
    v = m1_ref[...]
    ui = lax.bitcast_convert_type(v, jnp.int32)
    neg = lax.shift_right_arithmetic(ui, 31)
    key = lax.bitwise_xor(ui, lax.bitwise_and(neg, jnp.int32(0x7FFFFFFF)))

    def count_ge(t):
        return jnp.sum((key >= t).astype(jnp.int32), axis=1, keepdims=True)

    cnt_pos = count_ge(jnp.zeros((QT, 1), jnp.int32))
    t0 = jnp.where(cnt_pos >= K, jnp.int32(0), jnp.int32(-2147483648))

    def step(i, t):
        b = 30 - i
        cand = t + lax.shift_left(jnp.int32(1), b)
        return jnp.where(count_ge(cand) >= K, cand, t)

    t = lax.fori_loop(0, 31, step, t0)
    tneg = lax.shift_right_arithmetic(t, 31)
    tui = lax.bitwise_xor(t, lax.bitwise_and(tneg, jnp.int32(0x7FFFFFFF)))
    thr_ref[...] = lax.bitcast_convert_type(tui, jnp.float32)


def _thresholds(m1):
    return pl.pallas_call(
        _thr_body,
        grid=(BATCH // QT,),
        in_specs=[pl.BlockSpec((QT, NBLK), lambda i: (i, 0))],
        out_specs=pl.BlockSpec((QT, 1), lambda i: (i, 0)),
        out_shape=jax.ShapeDtypeStruct((BATCH, 1), jnp.float32),
    )(m1)


def _iota16():
    return lax.iota(jnp.int32, 16)


def _splat16(x):
    return jnp.zeros((16,), jnp.int32) + x


def _scalar(v):
    return jnp.max(v)


def _compact_body(m1_hbm, thr_hbm, sb_hbm, cs_hbm, ci_hbm,
                  m1_v, thr_v, bid_v, gid_v, rows_v, cs_v, ci_v, sem):
    cid = lax.axis_index("c")
    sid = lax.axis_index("s")
    wid = sid * 2 + cid
    base = wid * ROWS_PER_W
    pltpu.sync_copy(thr_hbm.at[pl.ds(base, ROWS_PER_W)], thr_v)

    def row_body(r_local, carry):
        r = base + r_local
        pltpu.sync_copy(m1_hbm.at[r], m1_v)
        tvec_i = plsc.load_gather(thr_v, [_splat16(r_local)])
        tvec = plsc.bitcast(tvec_i, jnp.float32)

        for t in range(9):
            bid_v[pl.ds(t * 16, 16)] = jnp.zeros((16,), jnp.int32)
        for t in range(NCAND // 16 + 1):
            cs_v[pl.ds(t * 16, 16)] = jnp.full((16,), NEG_INF)
            ci_v[pl.ds(t * 16, 16)] = jnp.zeros((16,), jnp.int32)

        cnt = jnp.int32(0)
        for c in range(NBLK // 16):
            mv = m1_v[pl.ds(c * 16, 16)]
            msk = mv >= tvec
            ids = _iota16() + (c * 16)
            plsc.store_compressed(bid_v.at[pl.ds(cnt, 16)], ids, mask=msk)
            pc = _scalar(plsc.all_reduce_population_count(msk))
            cnt = jnp.minimum(cnt + pc, 128)

        for t in range(8):
            gid_v[pl.ds(t * 16, 16)] = bid_v[pl.ds(t * 16, 16)] + r * NBLK
        pltpu.async_copy(sb_hbm.at[gid_v], rows_v, sem).wait()

        def blk_body(j, ccnt):
            bvec = plsc.load_gather(bid_v, [_splat16(j)])
            for k in range(8):
                sv = plsc.load_gather(
                    rows_v, [_splat16(j), _iota16() + (k * 16)])
                msk = sv >= tvec
                col = bvec * 128 + (k * 16) + _iota16()
                plsc.store_compressed(cs_v.at[pl.ds(ccnt, 16)], sv, mask=msk)
                plsc.store_compressed(ci_v.at[pl.ds(ccnt, 16)], col, mask=msk)
                pc = _scalar(plsc.all_reduce_population_count(msk))
                ccnt = jnp.minimum(ccnt + pc, NCAND)
            return ccnt

        lax.fori_loop(0, cnt, blk_body, jnp.int32(0))

        pltpu.sync_copy(cs_v.at[pl.ds(0, NCAND)], cs_hbm.at[r])
        pltpu.sync_copy(ci_v.at[pl.ds(0, NCAND)], ci_hbm.at[r])
        return carry

    lax.fori_loop(0, ROWS_PER_W, row_body, jnp.int32(0))


def _compact_candidates(m1_rows, thr, score_blocks):
    mesh = plsc.VectorSubcoreMesh(core_axis_name="c", subcore_axis_name="s")
    thr_i = lax.bitcast_convert_type(thr, jnp.int32)
    f = pl.kernel(
        _compact_body,
        out_type=[
            jax.ShapeDtypeStruct((BATCH, NCAND), jnp.float32),
            jax.ShapeDtypeStruct((BATCH, NCAND), jnp.int32),
        ],
        mesh=mesh,
        compiler_params=pltpu.CompilerParams(needs_layout_passes=False),
        scratch_types=[
            pltpu.VMEM((NBLK,), jnp.float32),
            pltpu.VMEM((ROWS_PER_W,), jnp.int32),
            pltpu.VMEM((144,), jnp.int32),
            pltpu.VMEM((128,), jnp.int32),
            pltpu.VMEM((128, 128), jnp.float32),
            pltpu.VMEM((NCAND + 16,), jnp.float32),
            pltpu.VMEM((NCAND + 16,), jnp.int32),
            pltpu.SemaphoreType.DMA,
        ],
    )
    return f(m1_rows, thr_i, score_blocks)


def _topk_body(cs_ref, ci_ref, os_ref, oi_ref):
    v0 = cs_ref[...]
    ci = ci_ref[...]
    lane = lax.broadcasted_iota(jnp.int32, (QT_D, 128), 1)
    out_s0 = jnp.full((QT_D, 128), NEG_INF)
    out_i0 = jnp.zeros((QT_D, 128), jnp.int32)

    def step(i, carry):
        v, out_s, out_i = carry
        m = jnp.max(v, axis=1, keepdims=True)
        sel = v == m
        idxm = jnp.min(jnp.where(sel, ci, jnp.int32(2147483647)),
                       axis=1, keepdims=True)
        v = jnp.where(sel & (ci == idxm), NEG_INF, v)
        out_s = jnp.where(lane == i, m, out_s)
        out_i = jnp.where(lane == i, idxm, out_i)
        return v, out_s, out_i

    _, out_s, out_i = lax.fori_loop(0, K, step, (v0, out_s0, out_i0))
    os_ref[...] = out_s
    oi_ref[...] = out_i


def _topk_of_candidates(cs, ci):
    return pl.pallas_call(
        _topk_body,
        grid=(BATCH // QT_D,),
        in_specs=[
            pl.BlockSpec((QT_D, NCAND), lambda i: (i, 0)),
            pl.BlockSpec((QT_D, NCAND), lambda i: (i, 0)),
        ],
        out_specs=[
            pl.BlockSpec((QT_D, 128), lambda i: (i, 0)),
            pl.BlockSpec((QT_D, 128), lambda i: (i, 0)),
        ],
        out_shape=[
            jax.ShapeDtypeStruct((BATCH, 128), jnp.float32),
            jax.ShapeDtypeStruct((BATCH, 128), jnp.int32),
        ],
    )(cs, ci)


def _gather_body(idx_hbm, corpus_hbm, out_hbm, idx_v, emb_v, sem):
    cid = lax.axis_index("c")
    sid = lax.axis_index("s")
    wid = sid * 2 + cid
    base = wid * ROWS_PER_W

    def row_body(r_local, carry):
        r = base + r_local
        pltpu.sync_copy(idx_hbm.at[r], idx_v)
        pltpu.async_copy(corpus_hbm.at[idx_v], emb_v, sem).wait()
        pltpu.sync_copy(emb_v.at[pl.ds(0, K)], out_hbm.at[r])
        return carry

    lax.fori_loop(0, ROWS_PER_W, row_body, jnp.int32(0))


def _gather_embeddings(idx_padded, corpus):
    mesh = plsc.VectorSubcoreMesh(core_axis_name="c", subcore_axis_name="s")
    f = pl.kernel(
        _gather_body,
        out_type=jax.ShapeDtypeStruct((BATCH, K, DIM), jnp.float32),
        mesh=mesh,
        compiler_params=pltpu.CompilerParams(needs_layout_passes=False),
        scratch_types=[
            pltpu.VMEM((104,), jnp.int32),
            pltpu.VMEM((104, DIM), jnp.float32),
            pltpu.SemaphoreType.DMA,
        ],
    )
    return f(idx_padded, corpus)


def kernel(query_embedding, corpus, num_items):
    corpus_padded = jnp.pad(corpus, ((0, CPAD - CORPUS), (0, 0)))
    scores, m1 = _scores_and_blockmax(query_embedding, corpus_padded)
    m1_rows = jnp.transpose(m1, (1, 0, 2)).reshape(BATCH, NBLK)
    thr = _thresholds(m1_rows)
    score_blocks = scores.reshape(BATCH * NBLK, 128)
    cs, ci = _compact_candidates(m1_rows, thr.reshape(BATCH), score_blocks)
    out_s, out_i = _topk_of_candidates(cs, ci)
    indices = out_i[:, :K] + (num_items - K)
    mips_scores = out_s[:, :K]
    idx_padded = jnp.pad(indices, ((0, 0), (0, 104 - K)))
    embeddings = _gather_embeddings(idx_padded, corpus)
    return (indices, mips_scores, embeddings)
